# baseline (speedup 1.0000x reference)
import functools

import jax
import jax.numpy as jnp
from jax import lax
from jax.experimental import pallas as pl
from jax.experimental.pallas import tpu as pltpu
from jax.experimental.pallas import tpu_sc as plsc

_BATCH = 16384
_ED = 64
_N_ACT = 128
_PAIR = 2 * _ED

_NC = 2
_NS = 16
_NW = _NC * _NS
_RPW = _BATCH // _NW
_CW = 128
_IDXR = _RPW // _CW
_GRP = 16


def _sc_gather_body(idx_hbm, tab, out_hbm, idx_v, stage, sem):
    wid = lax.axis_index("s") * _NC + lax.axis_index("c")
    crow = wid * _IDXR
    base = wid * _RPW
    pltpu.sync_copy(idx_hbm.at[pl.ds(crow, _IDXR)], idx_v)

    def grp_body(g, _):
        vec = idx_v[g >> 3, pl.ds((g & 7) * _GRP, _GRP)]
        copies = [
            pltpu.async_copy(
                tab.at[pl.ds(vec[k] >> 1, 1)],
                stage.at[pl.ds(g * _GRP + k, 1)],
                sem,
            )
            for k in range(_GRP)
        ]
        for c in copies:
            c.wait()
        return 0

    lax.fori_loop(0, _RPW // _GRP, grp_body, 0)
    pltpu.sync_copy(stage, out_hbm.at[pl.ds(base, _RPW)])


@functools.cache
def _sc_gather():
    return functools.partial(
        pl.kernel,
        out_type=jax.ShapeDtypeStruct((_BATCH, _PAIR), jnp.float32),
        mesh=plsc.VectorSubcoreMesh(core_axis_name="c", subcore_axis_name="s"),
        scratch_types=[
            pltpu.VMEM((_IDXR, _CW), jnp.int32),
            pltpu.VMEM((_RPW, _PAIR), jnp.float32),
            pltpu.SemaphoreType.DMA,
        ],
    )(_sc_gather_body)


_BLK = 2048


def _mlp_body(up_ref, ip_ref, uh_ref, ih_ref, w1u2_ref, w1i2_ref, b1_ref,
              w2_ref, b2_ref, o_ref):
    lane = lax.broadcasted_iota(jnp.int32, (_BLK, _PAIR), 1).astype(jnp.float32)
    um = jnp.where((lane >= _ED) == (uh_ref[...] > 0.5), up_ref[...], 0.0)
    im = jnp.where((lane >= _ED) == (ih_ref[...] > 0.5), ip_ref[...], 0.0)
    h = jnp.dot(um, w1u2_ref[...], preferred_element_type=jnp.float32)
    h += jnp.dot(im, w1i2_ref[...], preferred_element_type=jnp.float32)
    h = jnp.maximum(h + b1_ref[...], 0.0)
    p = jnp.sum(h * w2_ref[...], axis=1, keepdims=True) + b2_ref[...]
    o_ref[...] = 5.0 / (1.0 + jnp.exp(-p))


_mlp = pl.pallas_call(
    _mlp_body,
    grid=(_BATCH // _BLK,),
    in_specs=[
        pl.BlockSpec((_BLK, _PAIR), lambda i: (i, 0)),
        pl.BlockSpec((_BLK, _PAIR), lambda i: (i, 0)),
        pl.BlockSpec((_BLK, 1), lambda i: (i, 0)),
        pl.BlockSpec((_BLK, 1), lambda i: (i, 0)),
        pl.BlockSpec((_PAIR, _N_ACT), lambda i: (0, 0)),
        pl.BlockSpec((_PAIR, _N_ACT), lambda i: (0, 0)),
        pl.BlockSpec((1, _N_ACT), lambda i: (0, 0)),
        pl.BlockSpec((1, _N_ACT), lambda i: (0, 0)),
        pl.BlockSpec((1, 1), lambda i: (0, 0)),
    ],
    out_specs=pl.BlockSpec((_BLK, 1), lambda i: (i, 0)),
    out_shape=jax.ShapeDtypeStruct((_BATCH, 1), jnp.float32),
)


def kernel(X, user_table, item_table, W1, b1, W2, b2):
    Xi = X.astype(jnp.int32)
    u_idx = Xi[:, 0].reshape(_BATCH // _CW, _CW)
    i_idx = Xi[:, 1].reshape(_BATCH // _CW, _CW)
    uh = (Xi[:, 0] & 1).astype(jnp.float32).reshape(_BATCH, 1)
    ih = (Xi[:, 1] & 1).astype(jnp.float32).reshape(_BATCH, 1)
    upk = user_table.reshape(500000, _PAIR)
    ipk = item_table.reshape(500000, _PAIR)
    u_pairs = _sc_gather()(u_idx, upk)
    i_pairs = _sc_gather()(i_idx, ipk)
    w1u2 = jnp.concatenate([W1[:_ED], W1[:_ED]], axis=0)
    w1i2 = jnp.concatenate([W1[_ED:], W1[_ED:]], axis=0)
    return _mlp(u_pairs, i_pairs, uh, ih, w1u2, w1i2,
                b1.reshape(1, _N_ACT), W2.reshape(1, _N_ACT),
                b2.reshape(1, 1))


# speedup vs baseline: 1.5957x; 1.5957x over previous
import functools

import jax
import jax.numpy as jnp
from jax import lax
from jax.experimental import pallas as pl
from jax.experimental.pallas import tpu as pltpu
from jax.experimental.pallas import tpu_sc as plsc

_BATCH = 16384
_ED = 64
_N_ACT = 128
_PAIR = 2 * _ED
_NROWS = 1000000
_TCOLS = 4096
_TGRID = -(-_NROWS // _TCOLS)
_PROWS = _TGRID * (_TCOLS // 2)

_NC = 2
_NS = 16
_NW = _NC * _NS
_RPW = _BATCH // _NW
_CW = 128
_IDXR = _RPW // _CW
_GRP = 16


def _sc_gather_body(idx_hbm, tab, out_hbm, idx_v, stage, sem):
    wid = lax.axis_index("s") * _NC + lax.axis_index("c")
    crow = wid * _IDXR
    base = wid * _RPW
    pltpu.sync_copy(idx_hbm.at[pl.ds(crow, _IDXR)], idx_v)

    def grp_body(g, _):
        vec = idx_v[g >> 3, pl.ds((g & 7) * _GRP, _GRP)]
        tvec = ((vec >> 12) << 11) | (vec & 2047)
        copies = [
            pltpu.async_copy(
                tab.at[pl.ds(tvec[k], 1)],
                stage.at[pl.ds(g * _GRP + k, 1)],
                sem,
            )
            for k in range(_GRP)
        ]
        for c in copies:
            c.wait()
        return 0

    lax.fori_loop(0, _RPW // _GRP, grp_body, 0)
    pltpu.sync_copy(stage, out_hbm.at[pl.ds(base, _RPW)])


@functools.cache
def _sc_gather():
    return functools.partial(
        pl.kernel,
        out_type=jax.ShapeDtypeStruct((_BATCH, _PAIR), jnp.float32),
        mesh=plsc.VectorSubcoreMesh(core_axis_name="c", subcore_axis_name="s"),
        scratch_types=[
            pltpu.VMEM((_IDXR, _CW), jnp.int32),
            pltpu.VMEM((_RPW, _PAIR), jnp.float32),
            pltpu.SemaphoreType.DMA,
        ],
    )(_sc_gather_body)


def _pack_body(a_ref, o_ref):
    xT = a_ref[...].T
    o_ref[...] = jnp.concatenate(
        [xT[: _TCOLS // 2], xT[_TCOLS // 2:]], axis=1)


_pack = pl.pallas_call(
    _pack_body,
    grid=(_TGRID,),
    in_specs=[pl.BlockSpec((_ED, _TCOLS), lambda i: (0, i))],
    out_specs=pl.BlockSpec((_TCOLS // 2, _PAIR), lambda i: (i, 0)),
    out_shape=jax.ShapeDtypeStruct((_PROWS, _PAIR), jnp.float32),
)


def _pack_table(table):
    return _pack(table.T)


_BLK = 2048


def _mlp_body(up_ref, ip_ref, uh_ref, ih_ref, w1u2_ref, w1i2_ref, b1_ref,
              w2_ref, b2_ref, o_ref):
    lane = lax.broadcasted_iota(jnp.int32, (_BLK, _PAIR), 1).astype(jnp.float32)
    um = jnp.where((lane >= _ED) == (uh_ref[...] > 0.5), up_ref[...], 0.0)
    im = jnp.where((lane >= _ED) == (ih_ref[...] > 0.5), ip_ref[...], 0.0)
    h = jnp.dot(um, w1u2_ref[...], preferred_element_type=jnp.float32)
    h += jnp.dot(im, w1i2_ref[...], preferred_element_type=jnp.float32)
    h = jnp.maximum(h + b1_ref[...], 0.0)
    p = jnp.sum(h * w2_ref[...], axis=1, keepdims=True) + b2_ref[...]
    o_ref[...] = 5.0 / (1.0 + jnp.exp(-p))


_mlp = pl.pallas_call(
    _mlp_body,
    grid=(_BATCH // _BLK,),
    in_specs=[
        pl.BlockSpec((_BLK, _PAIR), lambda i: (i, 0)),
        pl.BlockSpec((_BLK, _PAIR), lambda i: (i, 0)),
        pl.BlockSpec((_BLK, 1), lambda i: (i, 0)),
        pl.BlockSpec((_BLK, 1), lambda i: (i, 0)),
        pl.BlockSpec((_PAIR, _N_ACT), lambda i: (0, 0)),
        pl.BlockSpec((_PAIR, _N_ACT), lambda i: (0, 0)),
        pl.BlockSpec((1, _N_ACT), lambda i: (0, 0)),
        pl.BlockSpec((1, _N_ACT), lambda i: (0, 0)),
        pl.BlockSpec((1, 1), lambda i: (0, 0)),
    ],
    out_specs=pl.BlockSpec((_BLK, 1), lambda i: (i, 0)),
    out_shape=jax.ShapeDtypeStruct((_BATCH, 1), jnp.float32),
)


def kernel(X, user_table, item_table, W1, b1, W2, b2):
    Xi = X.astype(jnp.int32)
    u_idx = Xi[:, 0].reshape(_BATCH // _CW, _CW)
    i_idx = Xi[:, 1].reshape(_BATCH // _CW, _CW)
    uh = ((Xi[:, 0] >> 11) & 1).astype(jnp.float32).reshape(_BATCH, 1)
    ih = ((Xi[:, 1] >> 11) & 1).astype(jnp.float32).reshape(_BATCH, 1)
    upk = _pack_table(user_table)
    ipk = _pack_table(item_table)
    u_pairs = _sc_gather()(u_idx, upk)
    i_pairs = _sc_gather()(i_idx, ipk)
    w1u2 = jnp.concatenate([W1[:_ED], W1[:_ED]], axis=0)
    w1i2 = jnp.concatenate([W1[_ED:], W1[_ED:]], axis=0)
    return _mlp(u_pairs, i_pairs, uh, ih, w1u2, w1i2,
                b1.reshape(1, _N_ACT), W2.reshape(1, _N_ACT),
                b2.reshape(1, 1))


# speedup vs baseline: 1.9612x; 1.2291x over previous
import functools

import jax
import jax.numpy as jnp
from jax import lax
from jax.experimental import pallas as pl
from jax.experimental.pallas import tpu as pltpu
from jax.experimental.pallas import tpu_sc as plsc

_BATCH = 16384
_ED = 64
_N_ACT = 128
_PAIR = 2 * _ED
_NROWS = 1000000
_TCOLS = 8192
_TGRID = -(-_NROWS // _TCOLS)
_PROWS = _TGRID * (_TCOLS // 2)

_NC = 2
_NS = 16
_NW = _NC * _NS
_RPW = _BATCH // _NW
_CW = 128
_IDXR = _RPW // _CW
_GRP = 16


def _sc_gather_body(idx_hbm, tab, out_hbm, idx_v, stage, sem):
    wid = lax.axis_index("s") * _NC + lax.axis_index("c")
    crow = wid * _IDXR
    base = wid * _RPW
    pltpu.sync_copy(idx_hbm.at[pl.ds(crow, _IDXR)], idx_v)

    def grp_body(g, _):
        vec = idx_v[g >> 3, pl.ds((g & 7) * _GRP, _GRP)]
        tvec = ((vec >> 13) << 12) | (vec & 4095)
        copies = [
            pltpu.async_copy(
                tab.at[pl.ds(tvec[k], 1)],
                stage.at[pl.ds(g * _GRP + k, 1)],
                sem,
            )
            for k in range(_GRP)
        ]
        for c in copies:
            c.wait()
        return 0

    lax.fori_loop(0, _RPW // _GRP, grp_body, 0)
    pltpu.sync_copy(stage, out_hbm.at[pl.ds(base, _RPW)])


@functools.cache
def _sc_gather():
    return functools.partial(
        pl.kernel,
        out_type=jax.ShapeDtypeStruct((_BATCH, _PAIR), jnp.float32),
        mesh=plsc.VectorSubcoreMesh(core_axis_name="c", subcore_axis_name="s"),
        scratch_types=[
            pltpu.VMEM((_IDXR, _CW), jnp.int32),
            pltpu.VMEM((_RPW, _PAIR), jnp.float32),
            pltpu.SemaphoreType.DMA,
        ],
    )(_sc_gather_body)


def _pack_body(a_ref, o_ref):
    eye = jnp.eye(_ED, dtype=jnp.float32)
    a = a_ref[...]
    half = _TCOLS // 2
    o_ref[:, :_ED] = lax.dot_general(
        a[:, :half], eye, (((0,), (0,)), ((), ())),
        preferred_element_type=jnp.float32)
    o_ref[:, _ED:] = lax.dot_general(
        a[:, half:], eye, (((0,), (0,)), ((), ())),
        preferred_element_type=jnp.float32)


_pack = pl.pallas_call(
    _pack_body,
    grid=(_TGRID,),
    in_specs=[pl.BlockSpec((_ED, _TCOLS), lambda i: (0, i))],
    out_specs=pl.BlockSpec((_TCOLS // 2, _PAIR), lambda i: (i, 0)),
    out_shape=jax.ShapeDtypeStruct((_PROWS, _PAIR), jnp.float32),
)


def _pack_table(table):
    return _pack(table.T)


_BLK = 2048


def _mlp_body(up_ref, ip_ref, uh_ref, ih_ref, w1u2_ref, w1i2_ref, b1_ref,
              w2_ref, b2_ref, o_ref):
    lane = lax.broadcasted_iota(jnp.int32, (_BLK, _PAIR), 1).astype(jnp.float32)
    um = jnp.where((lane >= _ED) == (uh_ref[...] > 0.5), up_ref[...], 0.0)
    im = jnp.where((lane >= _ED) == (ih_ref[...] > 0.5), ip_ref[...], 0.0)
    h = jnp.dot(um, w1u2_ref[...], preferred_element_type=jnp.float32)
    h += jnp.dot(im, w1i2_ref[...], preferred_element_type=jnp.float32)
    h = jnp.maximum(h + b1_ref[...], 0.0)
    p = jnp.sum(h * w2_ref[...], axis=1, keepdims=True) + b2_ref[...]
    o_ref[...] = 5.0 / (1.0 + jnp.exp(-p))


_mlp = pl.pallas_call(
    _mlp_body,
    grid=(_BATCH // _BLK,),
    in_specs=[
        pl.BlockSpec((_BLK, _PAIR), lambda i: (i, 0)),
        pl.BlockSpec((_BLK, _PAIR), lambda i: (i, 0)),
        pl.BlockSpec((_BLK, 1), lambda i: (i, 0)),
        pl.BlockSpec((_BLK, 1), lambda i: (i, 0)),
        pl.BlockSpec((_PAIR, _N_ACT), lambda i: (0, 0)),
        pl.BlockSpec((_PAIR, _N_ACT), lambda i: (0, 0)),
        pl.BlockSpec((1, _N_ACT), lambda i: (0, 0)),
        pl.BlockSpec((1, _N_ACT), lambda i: (0, 0)),
        pl.BlockSpec((1, 1), lambda i: (0, 0)),
    ],
    out_specs=pl.BlockSpec((_BLK, 1), lambda i: (i, 0)),
    out_shape=jax.ShapeDtypeStruct((_BATCH, 1), jnp.float32),
)


def kernel(X, user_table, item_table, W1, b1, W2, b2):
    Xi = X.astype(jnp.int32)
    u_idx = Xi[:, 0].reshape(_BATCH // _CW, _CW)
    i_idx = Xi[:, 1].reshape(_BATCH // _CW, _CW)
    uh = ((Xi[:, 0] >> 12) & 1).astype(jnp.float32).reshape(_BATCH, 1)
    ih = ((Xi[:, 1] >> 12) & 1).astype(jnp.float32).reshape(_BATCH, 1)
    upk = _pack_table(user_table)
    ipk = _pack_table(item_table)
    u_pairs = _sc_gather()(u_idx, upk)
    i_pairs = _sc_gather()(i_idx, ipk)
    w1u2 = jnp.concatenate([W1[:_ED], W1[:_ED]], axis=0)
    w1i2 = jnp.concatenate([W1[_ED:], W1[_ED:]], axis=0)
    return _mlp(u_pairs, i_pairs, uh, ih, w1u2, w1i2,
                b1.reshape(1, _N_ACT), W2.reshape(1, _N_ACT),
                b2.reshape(1, 1))


# speedup vs baseline: 2.2052x; 1.1244x over previous
import functools

import jax
import jax.numpy as jnp
from jax import lax
from jax.experimental import pallas as pl
from jax.experimental.pallas import tpu as pltpu
from jax.experimental.pallas import tpu_sc as plsc

_BATCH = 16384
_ED = 64
_N_ACT = 128
_PAIR = 2 * _ED
_NROWS = 1000000
_TCOLS = 16384
_TGRID = -(-_NROWS // _TCOLS)
_PROWS = _TGRID * (_TCOLS // 2)

_NC = 2
_NS = 16
_NW = _NC * _NS
_RPW = _BATCH // _NW
_CW = 128
_IDXR = _RPW // _CW
_GRP = 16


def _sc_gather_body(idx_hbm, tab, out_hbm, idx_v, stage, sem):
    wid = lax.axis_index("s") * _NC + lax.axis_index("c")
    crow = wid * _IDXR
    base = wid * _RPW
    pltpu.sync_copy(idx_hbm.at[pl.ds(crow, _IDXR)], idx_v)

    def grp_body(g, _):
        vec = idx_v[g >> 3, pl.ds((g & 7) * _GRP, _GRP)]
        tvec = ((vec >> 14) << 13) | (vec & 8191)
        copies = [
            pltpu.async_copy(
                tab.at[pl.ds(tvec[k], 1)],
                stage.at[pl.ds(g * _GRP + k, 1)],
                sem,
            )
            for k in range(_GRP)
        ]
        for c in copies:
            c.wait()
        return 0

    lax.fori_loop(0, _RPW // _GRP, grp_body, 0)
    pltpu.sync_copy(stage, out_hbm.at[pl.ds(base, _RPW)])


@functools.cache
def _sc_gather():
    return functools.partial(
        pl.kernel,
        out_type=jax.ShapeDtypeStruct((_BATCH, _PAIR), jnp.float32),
        mesh=plsc.VectorSubcoreMesh(core_axis_name="c", subcore_axis_name="s"),
        scratch_types=[
            pltpu.VMEM((_IDXR, _CW), jnp.int32),
            pltpu.VMEM((_RPW, _PAIR), jnp.float32),
            pltpu.SemaphoreType.DMA,
        ],
    )(_sc_gather_body)


def _pack_body(a_ref, o_ref):
    eye = jnp.eye(_ED, dtype=jnp.float32)
    a = a_ref[...]
    half = _TCOLS // 2
    o_ref[:, :_ED] = lax.dot_general(
        a[:, :half], eye, (((0,), (0,)), ((), ())),
        preferred_element_type=jnp.float32)
    o_ref[:, _ED:] = lax.dot_general(
        a[:, half:], eye, (((0,), (0,)), ((), ())),
        preferred_element_type=jnp.float32)


_pack = pl.pallas_call(
    _pack_body,
    grid=(_TGRID,),
    in_specs=[pl.BlockSpec((_ED, _TCOLS), lambda i: (0, i))],
    out_specs=pl.BlockSpec((_TCOLS // 2, _PAIR), lambda i: (i, 0)),
    out_shape=jax.ShapeDtypeStruct((_PROWS, _PAIR), jnp.float32),
)


def _pack_table(table):
    return _pack(table.T)


_BLK = 2048


def _mlp_body(up_ref, ip_ref, uh_ref, ih_ref, w1u2_ref, w1i2_ref, b1_ref,
              w2_ref, b2_ref, o_ref):
    lane = lax.broadcasted_iota(jnp.int32, (_BLK, _PAIR), 1).astype(jnp.float32)
    um = jnp.where((lane >= _ED) == (uh_ref[...] > 0.5), up_ref[...], 0.0)
    im = jnp.where((lane >= _ED) == (ih_ref[...] > 0.5), ip_ref[...], 0.0)
    h = jnp.dot(um, w1u2_ref[...], preferred_element_type=jnp.float32)
    h += jnp.dot(im, w1i2_ref[...], preferred_element_type=jnp.float32)
    h = jnp.maximum(h + b1_ref[...], 0.0)
    p = jnp.sum(h * w2_ref[...], axis=1, keepdims=True) + b2_ref[...]
    o_ref[...] = 5.0 / (1.0 + jnp.exp(-p))


_mlp = pl.pallas_call(
    _mlp_body,
    grid=(_BATCH // _BLK,),
    in_specs=[
        pl.BlockSpec((_BLK, _PAIR), lambda i: (i, 0)),
        pl.BlockSpec((_BLK, _PAIR), lambda i: (i, 0)),
        pl.BlockSpec((_BLK, 1), lambda i: (i, 0)),
        pl.BlockSpec((_BLK, 1), lambda i: (i, 0)),
        pl.BlockSpec((_PAIR, _N_ACT), lambda i: (0, 0)),
        pl.BlockSpec((_PAIR, _N_ACT), lambda i: (0, 0)),
        pl.BlockSpec((1, _N_ACT), lambda i: (0, 0)),
        pl.BlockSpec((1, _N_ACT), lambda i: (0, 0)),
        pl.BlockSpec((1, 1), lambda i: (0, 0)),
    ],
    out_specs=pl.BlockSpec((_BLK, 1), lambda i: (i, 0)),
    out_shape=jax.ShapeDtypeStruct((_BATCH, 1), jnp.float32),
)


def kernel(X, user_table, item_table, W1, b1, W2, b2):
    Xi = X.astype(jnp.int32)
    u_idx = Xi[:, 0].reshape(_BATCH // _CW, _CW)
    i_idx = Xi[:, 1].reshape(_BATCH // _CW, _CW)
    uh = ((Xi[:, 0] >> 13) & 1).astype(jnp.float32).reshape(_BATCH, 1)
    ih = ((Xi[:, 1] >> 13) & 1).astype(jnp.float32).reshape(_BATCH, 1)
    upk = _pack_table(user_table)
    ipk = _pack_table(item_table)
    u_pairs = _sc_gather()(u_idx, upk)
    i_pairs = _sc_gather()(i_idx, ipk)
    w1u2 = jnp.concatenate([W1[:_ED], W1[:_ED]], axis=0)
    w1i2 = jnp.concatenate([W1[_ED:], W1[_ED:]], axis=0)
    return _mlp(u_pairs, i_pairs, uh, ih, w1u2, w1i2,
                b1.reshape(1, _N_ACT), W2.reshape(1, _N_ACT),
                b2.reshape(1, 1))
